# Initial kernel scaffold; baseline (speedup 1.0000x reference)
import jax
import jax.numpy as jnp
from jax.experimental import pallas as pl


def kernel(x, pos, edge_index, edge_attr, params, noises):
    raise NotImplementedError("write your pallas kernel here")


# speedup vs baseline: 1.0194x; 1.0194x over previous
import functools

import jax
import jax.numpy as jnp
from jax import lax
from jax.experimental import pallas as pl
from jax.experimental.pallas import tpu as pltpu
from jax.experimental.pallas import tpu_sc as plsc

_N = 10000
_E = 320000
_D = 128
_TD = 256
_CHUNK = 128
_NW = 32
_RSPAN = 632


def _mesh():
    return plsc.VectorSubcoreMesh(core_axis_name="c", subcore_axis_name="s")


def _sc_gather(table, src, dst):
    n_chunks = _E // _CHUNK
    iters = (n_chunks + _NW - 1) // _NW

    @functools.partial(
        pl.kernel,
        out_type=[
            jax.ShapeDtypeStruct((_E, _TD), jnp.float32),
            jax.ShapeDtypeStruct((_E, _TD), jnp.float32),
        ],
        mesh=_mesh(),
        scratch_types=[
            pltpu.MemorySpace.VMEM((_CHUNK,), jnp.int32),
            pltpu.MemorySpace.VMEM((_CHUNK,), jnp.int32),
            pltpu.MemorySpace.VMEM((_CHUNK, _TD), jnp.float32),
            pltpu.MemorySpace.VMEM((_CHUNK, _TD), jnp.float32),
            pltpu.SemaphoreType.DMA,
            pltpu.SemaphoreType.DMA,
        ],
    )
    def k(t_hbm, s_hbm, d_hbm, os_hbm, od_hbm, idx_s, idx_d, row_s, row_d,
          sem1, sem2):
        w = lax.axis_index("s") * 2 + lax.axis_index("c")

        def body(i, carry):
            c = w + _NW * i

            @pl.when(c < n_chunks)
            def _():
                base = c * _CHUNK
                pltpu.sync_copy(s_hbm.at[pl.ds(base, _CHUNK)], idx_s)
                pltpu.sync_copy(d_hbm.at[pl.ds(base, _CHUNK)], idx_d)
                cp1 = pltpu.async_copy(t_hbm.at[idx_s], row_s, sem1)
                cp2 = pltpu.async_copy(t_hbm.at[idx_d], row_d, sem2)
                cp1.wait()
                cp2.wait()
                pltpu.sync_copy(row_s, os_hbm.at[pl.ds(base, _CHUNK)])
                pltpu.sync_copy(row_d, od_hbm.at[pl.ds(base, _CHUNK)])

            return carry

        lax.fori_loop(0, iters, body, 0)

    return k(table, src, dst)


def _sc_scatter(msg, dst):
    per_core = _E // 2
    chunks_per_core = per_core // _CHUNK
    iters = (chunks_per_core + 15) // 16

    @functools.partial(
        pl.kernel,
        out_type=jax.ShapeDtypeStruct((2, _N, _D), jnp.float32),
        mesh=_mesh(),
        scratch_types=[
            pltpu.MemorySpace.VMEM((_CHUNK,), jnp.int32),
            pltpu.MemorySpace.VMEM((_CHUNK, _D), jnp.float32),
            pltpu.MemorySpace.VMEM((64, _D), jnp.float32),
            pltpu.MemorySpace.VMEM_SHARED((_N, _D), jnp.float32),
            pltpu.SemaphoreType.DMA,
        ],
    )
    def k(m_hbm, d_hbm, out_hbm, idx_v, rows_v, zbuf, acc, sem):
        cid = lax.axis_index("c")
        sid = lax.axis_index("s")

        def run(acc):
            zero = jnp.zeros((16,), jnp.float32)

            def zb(t, carry):
                zbuf[t // 8, pl.ds((t % 8) * 16, 16)] = zero
                return carry

            lax.fori_loop(0, 64 * 8, zb, 0)
            base_row = jnp.minimum((sid * 625) // 8 * 8, _N - _RSPAN)

            def zcopy(b, carry):
                pltpu.sync_copy(zbuf, acc.at[pl.ds(base_row + b * 64, 64)])
                return carry

            lax.fori_loop(0, 9, zcopy, 0)
            pltpu.sync_copy(zbuf.at[pl.ds(0, 56)],
                            acc.at[pl.ds(base_row + 576, 56)])
            plsc.subcore_barrier()

            @pl.when(sid == 0)
            def _():
                def body(kk, carry):
                    base = cid * per_core + kk * _CHUNK
                    pltpu.sync_copy(d_hbm.at[pl.ds(base, _CHUNK)], idx_v)
                    pltpu.sync_copy(m_hbm.at[pl.ds(base, _CHUNK)], rows_v)
                    pltpu.sync_copy(rows_v, acc.at[idx_v], add=True)
                    return carry

                lax.fori_loop(0, chunks_per_core, body, 0)

            plsc.subcore_barrier()
            pltpu.sync_copy(acc.at[pl.ds(base_row, _RSPAN)],
                            out_hbm.at[cid, pl.ds(base_row, _RSPAN)])

        run(acc)

    return k(msg, dst)


def _sc_counts(dst):
    per_core = _E // 2
    chunks_per_core = per_core // _CHUNK
    iters = (chunks_per_core + 15) // 16

    @functools.partial(
        pl.kernel,
        out_type=jax.ShapeDtypeStruct((2, _N, 16), jnp.float32),
        mesh=_mesh(),
        scratch_types=[
            pltpu.MemorySpace.VMEM((_CHUNK,), jnp.int32),
            pltpu.MemorySpace.VMEM((_CHUNK, 16), jnp.float32),
            pltpu.MemorySpace.VMEM((64, 16), jnp.float32),
            pltpu.MemorySpace.VMEM_SHARED((_N, 16), jnp.float32),
            pltpu.SemaphoreType.DMA,
        ],
    )
    def k(d_hbm, out_hbm, idx_v, ones_v, zbuf, acc, sem):
        cid = lax.axis_index("c")
        sid = lax.axis_index("s")

        def run(acc):
            lanes = lax.iota(jnp.int32, 16)
            onehot = jnp.where(lanes == 0, 1.0, 0.0).astype(jnp.float32)
            zero = jnp.zeros((16,), jnp.float32)

            def fill(r, carry):
                ones_v[r, pl.ds(0, 16)] = onehot
                return carry

            lax.fori_loop(0, _CHUNK, fill, 0)

            def zb(r, carry):
                zbuf[r, pl.ds(0, 16)] = zero
                return carry

            lax.fori_loop(0, 64, zb, 0)
            base_row = jnp.minimum((sid * 625) // 8 * 8, _N - _RSPAN)

            def zcopy(b, carry):
                pltpu.sync_copy(zbuf, acc.at[pl.ds(base_row + b * 64, 64)])
                return carry

            lax.fori_loop(0, 9, zcopy, 0)
            pltpu.sync_copy(zbuf.at[pl.ds(0, 56)],
                            acc.at[pl.ds(base_row + 576, 56)])
            plsc.subcore_barrier()

            def body(i, carry):
                kk = sid + 16 * i

                @pl.when(kk < chunks_per_core)
                def _():
                    base = cid * per_core + kk * _CHUNK
                    pltpu.sync_copy(d_hbm.at[pl.ds(base, _CHUNK)], idx_v)
                    pltpu.sync_copy(ones_v, acc.at[idx_v], add=True)

                return carry

            lax.fori_loop(0, iters, body, 0)
            plsc.subcore_barrier()
            pltpu.sync_copy(acc.at[pl.ds(base_row, _RSPAN)],
                            out_hbm.at[cid, pl.ds(base_row, _RSPAN)])

        run(acc)

    return k(dst)


def _silu(v):
    return v * jax.nn.sigmoid(v)


_BE = 2000
_BN = 2000


def _tc_msg_mlp(ts, td, ea, w1at, w1bt, w1ct, w1d, b1, w2t, b2):
    def body(ts_ref, td_ref, ea_ref, w1at_ref, w1bt_ref, w1ct_ref, w1d_ref,
             b1_ref, w2t_ref, b2_ref, out_ref):
        dp = ts_ref[:, 128:144] - td_ref[:, 128:144]
        dist = jnp.sqrt(jnp.sum(dp * dp, axis=1, keepdims=True))
        h1 = jnp.dot(td_ref[:, 0:128], w1at_ref[...],
                     preferred_element_type=jnp.float32)
        h1 += jnp.dot(ts_ref[:, 0:128], w1bt_ref[...],
                      preferred_element_type=jnp.float32)
        h1 += jnp.dot(ea_ref[...], w1ct_ref[...],
                      preferred_element_type=jnp.float32)
        h1 += dist * w1d_ref[...] + b1_ref[...]
        out_ref[...] = jnp.dot(_silu(h1), w2t_ref[...],
                               preferred_element_type=jnp.float32) + b2_ref[...]

    full = lambda i: (0, 0)
    return pl.pallas_call(
        body,
        grid=(_E // _BE,),
        in_specs=[
            pl.BlockSpec((_BE, _TD), lambda i: (i, 0)),
            pl.BlockSpec((_BE, _TD), lambda i: (i, 0)),
            pl.BlockSpec((_BE, 4), lambda i: (i, 0)),
            pl.BlockSpec((128, 256), full),
            pl.BlockSpec((128, 256), full),
            pl.BlockSpec((4, 256), full),
            pl.BlockSpec((1, 256), full),
            pl.BlockSpec((1, 256), full),
            pl.BlockSpec((256, 128), full),
            pl.BlockSpec((1, 128), full),
        ],
        out_specs=pl.BlockSpec((_BE, _D), lambda i: (i, 0)),
        out_shape=jax.ShapeDtypeStruct((_E, _D), jnp.float32),
    )(ts, td, ea, w1at, w1bt, w1ct, w1d, b1, w2t, b2)


def _tc_combine(s, cnt, x, pos_pad, noise_pad, c1t, c1b, c2t, c2b):
    def body(s_ref, cnt_ref, x_ref, pos_ref, noi_ref, c1t_ref, c1b_ref,
             c2t_ref, c2b_ref, nx_ref, np_ref):
        cnt = cnt_ref[0] + cnt_ref[1]
        aggr = (s_ref[0] + s_ref[1]) / jnp.maximum(cnt, 1.0)
        nx_ref[...] = x_ref[...] + aggr
        t = _silu(jnp.dot(aggr, c1t_ref[...],
                          preferred_element_type=jnp.float32) + c1b_ref[...])
        cu = jnp.dot(t, c2t_ref[...],
                     preferred_element_type=jnp.float32) + c2b_ref[...]
        np_ref[...] = pos_ref[...] + cu * noi_ref[...]

    full = lambda i: (0, 0)
    return pl.pallas_call(
        body,
        grid=(_N // _BN,),
        in_specs=[
            pl.BlockSpec((2, _BN, _D), lambda i: (0, i, 0)),
            pl.BlockSpec((2, _BN, 1), lambda i: (0, i, 0)),
            pl.BlockSpec((_BN, _D), lambda i: (i, 0)),
            pl.BlockSpec((_BN, 128), lambda i: (i, 0)),
            pl.BlockSpec((_BN, 128), lambda i: (i, 0)),
            pl.BlockSpec((128, 128), full),
            pl.BlockSpec((1, 128), full),
            pl.BlockSpec((128, 1), full),
            pl.BlockSpec((1, 1), full),
        ],
        out_specs=[
            pl.BlockSpec((_BN, _D), lambda i: (i, 0)),
            pl.BlockSpec((_BN, 128), lambda i: (i, 0)),
        ],
        out_shape=[
            jax.ShapeDtypeStruct((_N, _D), jnp.float32),
            jax.ShapeDtypeStruct((_N, 128), jnp.float32),
        ],
    )(s, cnt, x, pos_pad, noise_pad, c1t, c1b, c2t, c2b)


def _tc_emb(x, wt, b):
    def body(x_ref, w_ref, b_ref, o_ref):
        o_ref[...] = jnp.dot(x_ref[...], w_ref[...],
                             preferred_element_type=jnp.float32) + b_ref[...]

    full = lambda i: (0, 0)
    return pl.pallas_call(
        body,
        grid=(_N // _BN,),
        in_specs=[
            pl.BlockSpec((_BN, _D), lambda i: (i, 0)),
            pl.BlockSpec((128, 128), full),
            pl.BlockSpec((1, 128), full),
        ],
        out_specs=pl.BlockSpec((_BN, _D), lambda i: (i, 0)),
        out_shape=jax.ShapeDtypeStruct((_N, _D), jnp.float32),
    )(x, wt, b)


def _tc_readout(h, o1t, b1, o2t, b2):
    def body(h_ref, o1t_ref, b1_ref, o2t_ref, b2_ref, out_ref):
        g = jnp.mean(h_ref[...], axis=0, keepdims=True)
        z = _silu(jnp.dot(g, o1t_ref[...],
                          preferred_element_type=jnp.float32) + b1_ref[...])
        out_ref[...] = jnp.dot(z, o2t_ref[...],
                               preferred_element_type=jnp.float32) + b2_ref[...]

    return pl.pallas_call(
        body,
        out_shape=jax.ShapeDtypeStruct((1, 1), jnp.float32),
    )(h, o1t, b1, o2t, b2)


def kernel(x, pos, edge_index, edge_attr, params, noises):
    src = edge_index[0]
    dst = edge_index[1]

    h = _tc_emb(x, params["emb"]["W"].T, params["emb"]["b"].reshape(1, _D))

    pos_pad = jnp.pad(pos, ((0, 0), (0, 125)))
    cnt = jax.ops.segment_sum(jnp.ones((_E, 1), jnp.float32), dst,
                              num_segments=_N).reshape(1, _N, 1)
    cnt = jnp.concatenate([cnt, jnp.zeros_like(cnt)], axis=0)

    for i in range(4):
        p = params["layers"][i]
        w1 = p["msg1"]["W"]
        table = jnp.concatenate([h, pos_pad], axis=1)
        ts, td = _sc_gather(table, src, dst)
        msg = _tc_msg_mlp(
            ts, td, edge_attr,
            w1[:, 0:128].T, w1[:, 128:256].T, w1[:, 256:260].T,
            w1[:, 260].reshape(1, 256), p["msg1"]["b"].reshape(1, 256),
            p["msg2"]["W"].T, p["msg2"]["b"].reshape(1, _D))
        s = _sc_scatter(msg, dst)
        noise_pad = jnp.pad(noises[i], ((0, 0), (0, 125)))
        h, pos_pad = _tc_combine(
            s, cnt, h, pos_pad, noise_pad,
            p["coord1"]["W"].T, p["coord1"]["b"].reshape(1, _D),
            p["coord2"]["W"].T, p["coord2"]["b"].reshape(1, 1))

    out = _tc_readout(
        h, params["out1"]["W"].T, params["out1"]["b"].reshape(1, 64),
        params["out2"]["W"].T, params["out2"]["b"].reshape(1, 1))
    return out.reshape(1)
